# Initial kernel scaffold; baseline (speedup 1.0000x reference)
import jax
import jax.numpy as jnp
from jax.experimental import pallas as pl


def kernel(x, pos, batch, c1_W1, c1_b1, c1_g, c1_be, c1_W2, c1_b2, c2_W1, c2_b1, c2_g, c2_be, c2_W2, c2_b2, c3_W1, c3_b1, c3_g, c3_be, c3_W2, c3_b2, m_W1, m_b1, m_W2, m_b2, m_W3, m_b3, m_W4, m_b4):
    raise NotImplementedError("write your pallas kernel here")


# speedup vs baseline: 3.7470x; 3.7470x over previous
import functools

import jax
import jax.numpy as jnp
from jax import lax
from jax.experimental import pallas as pl
from jax.experimental.pallas import tpu as pltpu
from jax.experimental.pallas import tpu_sc as plsc

NPTS = 8192
KNN = 20
KPAD = 32
NE = NPTS * KNN
NW = 32
CH = 128
QBLK = 256
F32 = jnp.float32


def _pq_body(f_ref, wa_ref, wb_ref, b1_ref, p_ref, q_ref):
    f = f_ref[...]
    wb = wb_ref[...]
    wd = wa_ref[...] - wb
    q_ref[...] = jnp.dot(f, wb, preferred_element_type=F32)
    p_ref[...] = jnp.dot(f, wd, preferred_element_type=F32) + b1_ref[...]


def _pq(featp, wa, wb, b1):
    fp = featp.shape[1]
    grid = NPTS // 512
    return pl.pallas_call(
        _pq_body,
        grid=(grid,),
        in_specs=[
            pl.BlockSpec((512, fp), lambda g: (g, 0)),
            pl.BlockSpec((fp, 64), lambda g: (0, 0)),
            pl.BlockSpec((fp, 64), lambda g: (0, 0)),
            pl.BlockSpec((1, 64), lambda g: (0, 0)),
        ],
        out_specs=[
            pl.BlockSpec((512, 64), lambda g: (g, 0)),
            pl.BlockSpec((512, 64), lambda g: (g, 0)),
        ],
        out_shape=[
            jax.ShapeDtypeStruct((NPTS, 64), F32),
            jax.ShapeDtypeStruct((NPTS, 64), F32),
        ],
    )(featp, wa, wb, b1)


def _knn_body(fj_ref, fit_ref, bc_ref, br_ref, out_ref, dt_ref, io_ref):
    fj = fj_ref[...]
    fit = fit_ref[...]
    sqj = jnp.sum(fj * fj, axis=1, keepdims=True)
    sqi = jnp.sum(fit * fit, axis=0, keepdims=True)
    d = sqj + sqi - 2.0 * jnp.dot(fj, fit, preferred_element_type=F32)
    d = jnp.where(bc_ref[...] != br_ref[...], 1e10, d)
    dt_ref[...] = d
    io_ref[...] = lax.broadcasted_iota(jnp.int32, (NPTS, QBLK), 0)

    def body(k, carry):
        dd = dt_ref[...]
        io = io_ref[...]
        m = jnp.min(dd, axis=0, keepdims=True)
        sel = jnp.min(jnp.where(dd <= m, io, NPTS), axis=0,
                      keepdims=True)
        out_ref[pl.ds(k, 1), :] = sel
        dt_ref[...] = jnp.where(io == sel, 3.0e10, dd)
        return carry

    lax.fori_loop(0, KNN, body, 0)


def _knn(featp, featpT, bc, br):
    fp = featp.shape[1]
    grid = NPTS // QBLK
    return pl.pallas_call(
        _knn_body,
        grid=(grid,),
        in_specs=[
            pl.BlockSpec((NPTS, fp), lambda g: (0, 0)),
            pl.BlockSpec((fp, QBLK), lambda g: (0, g)),
            pl.BlockSpec((NPTS, 1), lambda g: (0, 0)),
            pl.BlockSpec((1, QBLK), lambda g: (0, g)),
        ],
        out_specs=pl.BlockSpec((KPAD, QBLK), lambda g: (0, g)),
        out_shape=jax.ShapeDtypeStruct((KPAD, NPTS), jnp.int32),
        scratch_shapes=[
            pltpu.VMEM((NPTS, QBLK), F32),
            pltpu.VMEM((NPTS, QBLK), jnp.int32),
        ],
    )(featp, featpT, bc, br)


def _sc_gather_body(idx_hbm, q_hbm, out_hbm, idx_v, rows_v, sem):
    c = lax.axis_index("c")
    s = lax.axis_index("s")
    wid = s * 2 + c
    base = wid * (NE // NW)

    def body(t, carry):
        off = pl.multiple_of(base + t * CH, CH)
        pltpu.sync_copy(idx_hbm.at[pl.ds(off, CH)], idx_v)
        pltpu.async_copy(q_hbm.at[idx_v], rows_v, sem).wait()
        pltpu.sync_copy(rows_v, out_hbm.at[pl.ds(off, CH), :])
        return carry

    lax.fori_loop(0, NE // NW // CH, body, 0)


def _sc_gather(idx_flat, qtab):
    mesh = plsc.VectorSubcoreMesh(core_axis_name="c", subcore_axis_name="s")
    fn = functools.partial(
        pl.kernel,
        out_type=jax.ShapeDtypeStruct((NE, 64), F32),
        mesh=mesh,
        compiler_params=pltpu.CompilerParams(use_tc_tiling_on_sc=False),
        scratch_types=[
            pltpu.VMEM((CH,), jnp.int32),
            pltpu.VMEM((CH, 64), F32),
            pltpu.SemaphoreType.DMA,
        ],
    )(_sc_gather_body)
    return fn(idx_flat, qtab)


def _stats_body(e_ref, p_ref, o_ref):
    g = pl.program_id(0)

    @pl.when(g == 0)
    def _():
        o_ref[...] = jnp.zeros((8, 64), F32)

    z = e_ref[...] + p_ref[...]
    o_ref[0:1, :] += jnp.sum(z, axis=0, keepdims=True)
    o_ref[1:2, :] += jnp.sum(z * z, axis=0, keepdims=True)


def _stats(e, p):
    grid = NE // 4096
    return pl.pallas_call(
        _stats_body,
        grid=(grid,),
        in_specs=[
            pl.BlockSpec((4096, 64), lambda g: (g, 0)),
            pl.BlockSpec((4096, 64), lambda g: (g % 2, 0)),
        ],
        out_specs=pl.BlockSpec((8, 64), lambda g: (0, 0)),
        out_shape=jax.ShapeDtypeStruct((8, 64), F32),
    )(e, p)


def _edge_body(s_ref, p_ref, e_ref, g_ref, be_ref, w2_ref, b2_ref, o_ref):
    inv = 1.0 / float(NE)
    mu = s_ref[0:1, :] * inv
    var = s_ref[1:2, :] * inv - mu * mu
    a = g_ref[...] / jnp.sqrt(var + 1e-5)
    bb = be_ref[...] - mu * a
    p = p_ref[...]
    w2 = w2_ref[...]
    acc = None
    for k in range(KNN):
        z = jnp.maximum((p + e_ref[k]) * a + bb, 0.0)
        u = jnp.dot(z, w2, preferred_element_type=F32)
        acc = u if acc is None else jnp.maximum(acc, u)
    o_ref[...] = acc + b2_ref[...]


def _edge(sums, p, e3, gg, be, w2, b2):
    grid = NPTS // QBLK
    return pl.pallas_call(
        _edge_body,
        grid=(grid,),
        in_specs=[
            pl.BlockSpec((8, 64), lambda g: (0, 0)),
            pl.BlockSpec((QBLK, 64), lambda g: (g, 0)),
            pl.BlockSpec((KNN, QBLK, 64), lambda g: (0, g, 0)),
            pl.BlockSpec((1, 64), lambda g: (0, 0)),
            pl.BlockSpec((1, 64), lambda g: (0, 0)),
            pl.BlockSpec((64, 64), lambda g: (0, 0)),
            pl.BlockSpec((1, 64), lambda g: (0, 0)),
        ],
        out_specs=pl.BlockSpec((QBLK, 64), lambda g: (g, 0)),
        out_shape=jax.ShapeDtypeStruct((NPTS, 64), F32),
    )(sums, p, e3, gg, be, w2, b2)


def _head_body(x1_ref, x2_ref, x3_ref, w1_ref, b1_ref, w2_ref, b2_ref,
               w3_ref, b3_ref, w4_ref, b4_ref, o_ref):
    h = jnp.concatenate([x1_ref[...], x2_ref[...], x3_ref[...]], axis=1)
    h = jnp.maximum(jnp.dot(h, w1_ref[...], preferred_element_type=F32)
                    + b1_ref[...], 0.0)
    h = jnp.maximum(jnp.dot(h, w2_ref[...], preferred_element_type=F32)
                    + b2_ref[...], 0.0)
    h = jnp.maximum(jnp.dot(h, w3_ref[...], preferred_element_type=F32)
                    + b3_ref[...], 0.0)
    lg = jnp.dot(h, w4_ref[...], preferred_element_type=F32) + b4_ref[...]
    m = jnp.max(lg, axis=1, keepdims=True)
    lse = m + jnp.log(jnp.sum(jnp.exp(lg - m), axis=1, keepdims=True))
    o_ref[...] = lg - lse


def _head(x1, x2, x3, w1, b1, w2, b2, w3, b3, w4p, b4p):
    grid = NPTS // 512
    return pl.pallas_call(
        _head_body,
        grid=(grid,),
        in_specs=[
            pl.BlockSpec((512, 64), lambda g: (g, 0)),
            pl.BlockSpec((512, 64), lambda g: (g, 0)),
            pl.BlockSpec((512, 64), lambda g: (g, 0)),
            pl.BlockSpec((192, 1024), lambda g: (0, 0)),
            pl.BlockSpec((1, 1024), lambda g: (0, 0)),
            pl.BlockSpec((1024, 256), lambda g: (0, 0)),
            pl.BlockSpec((1, 256), lambda g: (0, 0)),
            pl.BlockSpec((256, 128), lambda g: (0, 0)),
            pl.BlockSpec((1, 128), lambda g: (0, 0)),
            pl.BlockSpec((128, 128), lambda g: (0, 0)),
            pl.BlockSpec((1, 128), lambda g: (0, 0)),
        ],
        out_specs=pl.BlockSpec((512, 128), lambda g: (g, 0)),
        out_shape=jax.ShapeDtypeStruct((NPTS, 128), F32),
    )(x1, x2, x3, w1, b1, w2, b2, w3, b3, w4p, b4p)


def _layer(featp, bc, br, W1, b1, gg, be, W2, b2):
    f = W1.shape[0] // 2
    fp = featp.shape[1]
    wa = W1[:f]
    wb = W1[f:]
    if f < fp:
        wa = jnp.pad(wa, ((0, fp - f), (0, 0)))
        wb = jnp.pad(wb, ((0, fp - f), (0, 0)))
    p, q = _pq(featp, wa, wb, b1[None, :])
    idxt = _knn(featp, featp.T, bc, br)
    idx_flat = idxt[:KNN].reshape(-1)
    e = _sc_gather(idx_flat, q)
    sums = _stats(e, p)
    e3 = e.reshape(KNN, NPTS, 64)
    return _edge(sums, p, e3, gg[None, :], be[None, :], W2, b2[None, :])


def kernel(x, pos, batch, c1_W1, c1_b1, c1_g, c1_be, c1_W2, c1_b2,
           c2_W1, c2_b1, c2_g, c2_be, c2_W2, c2_b2,
           c3_W1, c3_b1, c3_g, c3_be, c3_W2, c3_b2,
           m_W1, m_b1, m_W2, m_b2, m_W3, m_b3, m_W4, m_b4):
    batch_f = batch.astype(F32)
    bc = batch_f[:, None]
    br = batch_f[None, :]
    x0 = jnp.pad(jnp.concatenate([x, pos], axis=1), ((0, 0), (0, 2)))
    x1 = _layer(x0, bc, br, c1_W1, c1_b1, c1_g, c1_be, c1_W2, c1_b2)
    x2 = _layer(x1, bc, br, c2_W1, c2_b1, c2_g, c2_be, c2_W2, c2_b2)
    x3 = _layer(x2, bc, br, c3_W1, c3_b1, c3_g, c3_be, c3_W2, c3_b2)
    w4p = jnp.pad(m_W4, ((0, 0), (0, 88)))
    b4p = jnp.pad(m_b4, (0, 88), constant_values=-1e30)[None, :]
    lg = _head(x1, x2, x3, m_W1, m_b1[None, :], m_W2, m_b2[None, :],
               m_W3, m_b3[None, :], w4p, b4p)
    return lg[:, :40]
